# baseline (speedup 1.0000x reference)
import functools

import jax
import jax.numpy as jnp
from jax import lax
from jax.experimental import pallas as pl
from jax.experimental.pallas import tpu as pltpu
from jax.experimental.pallas import tpu_sc as plsc

B, N = 4, 32768
M = B * N
GH, GW = 256, 256
T = 16
FIN = 10
C = 32
Ptot = B * GH * GW
VX = 1.0 / GW
VY = 1.0 / GH
ZC = 0.5
EPS = 1e-5

NW = 32
CHUNK = M // NW
NBINS = B * GH
BIN_PAD = 1040
CAP = M + 704
PB = 8192

_mesh = plsc.VectorSubcoreMesh(core_axis_name="c", subcore_axis_name="s")
_sc_params = pltpu.CompilerParams(
    needs_layout_passes=False, use_tc_tiling_on_sc=False)


def _dup_ranks(v):
  lane = lax.broadcasted_iota(jnp.int32, (16,), 0)
  one = jnp.full((16,), 1, jnp.int32)
  zero = jnp.zeros((16,), jnp.int32)
  rank = jnp.zeros((16,), jnp.int32)
  total = jnp.zeros((16,), jnp.int32)
  for j in range(16):
    eqm = v == v[j]
    rank = rank + jnp.where(eqm & (lane > j), one, zero)
    total = total + jnp.where(eqm, one, zero)
  return rank, rank == total - 1


def _k1_body(pts_ref, out_ref):
  x = pts_ref[3:4, :]
  y = pts_ref[4:5, :]
  z = pts_ref[5:6, :]
  gi = (x * GW).astype(jnp.int32)
  gj = (y * GH).astype(jnp.int32)
  valid = (gi >= 0) & (gi < GW) & (gj >= 0) & (gj < GH) & (z >= 0.0) & (z < 1.0)
  rgb = pts_ref[0:3, :]
  out_ref[...] = jnp.max(jnp.where(valid, rgb, -jnp.inf)).reshape(1, 1)


def _k2_body(pts_ref, w_ref, rmax_ref, pid_ref, xlin_ref):
  i = pl.program_id(0)
  x = pts_ref[3:4, :]
  y = pts_ref[4:5, :]
  z = pts_ref[5:6, :]
  gi = (x * GW).astype(jnp.int32)
  gj = (y * GH).astype(jnp.int32)
  valid = (gi >= 0) & (gi < GW) & (gj >= 0) & (gj < GH) & (z >= 0.0) & (z < 1.0)
  gidx = i * PB + lax.broadcasted_iota(jnp.int32, (1, PB), 1)
  bidx = lax.shift_right_logical(gidx, 15)
  pid = jnp.where(valid, bidx * (GH * GW) + gj * GW + gi, Ptot)
  pid_ref[...] = pid.reshape(PB)

  scale = jnp.where(rmax_ref[0, 0] > 1.1, jnp.float32(1.0 / 255.0),
                    jnp.float32(1.0))
  cxp = (gi.astype(jnp.float32) + 0.5) * VX
  cyp = (gj.astype(jnp.float32) + 0.5) * VY
  r = pts_ref[0:1, :] * scale
  g = pts_ref[1:2, :] * scale
  bch = pts_ref[2:3, :] * scale
  ones = jnp.ones((1, PB), jnp.float32)
  featsT = jnp.concatenate(
      [x, y, z, r, g, bch, x - cxp, y - cyp, z - ZC, ones], axis=0)
  feats = jnp.transpose(featsT)
  xlin_ref[...] = lax.dot_general(
      feats, w_ref[...], (((1,), (1,)), ((), ())),
      preferred_element_type=jnp.float32)


def _k3_body(pid_hbm, hist_hbm, pid_v, hist_v):
  wid = lax.axis_index("s") * 2 + lax.axis_index("c")
  pltpu.sync_copy(pid_hbm.at[pl.ds(pl.multiple_of(wid * CHUNK, 8), CHUNK)], pid_v)
  sh8 = jnp.full((16,), 8, jnp.int32)

  def zero(i, _):
    hist_v[pl.ds(i * 16, 16)] = jnp.zeros((16,), jnp.int32)
    return 0
  lax.fori_loop(0, BIN_PAD // 16, zero, 0)

  def step(t, _):
    v = pid_v[pl.ds(t * 16, 16)]
    bins = lax.shift_right_logical(v, sh8)
    rank, last = _dup_ranks(bins)
    cur = plsc.load_gather(hist_v, [bins])
    plsc.store_scatter(hist_v, [bins], cur + rank + 1, mask=last)
    return 0
  lax.fori_loop(0, CHUNK // 16, step, 0)
  pltpu.sync_copy(hist_v, hist_hbm.at[wid])


def _k4_body(pid_hbm, hist_hbm, pids_out, idxs_out,
             pid_v, hist_v, off_v, dst_v, idx_v, sem):
  wid = lax.axis_index("s") * 2 + lax.axis_index("c")
  lane = lax.broadcasted_iota(jnp.int32, (16,), 0)
  pltpu.sync_copy(pid_hbm.at[pl.ds(pl.multiple_of(wid * CHUNK, 8), CHUNK)], pid_v)
  pltpu.sync_copy(hist_hbm, hist_v)
  sh8 = jnp.full((16,), 8, jnp.int32)

  @pl.when(wid == 0)
  def _():
    def z(i, _):
      idx_v[pl.ds(i * 16, 16)] = jnp.zeros((16,), jnp.int32)
      return 0
    lax.fori_loop(0, (CAP - M) // 16, z, 0)
    pltpu.sync_copy(idx_v.at[pl.ds(0, CAP - M)],
                    idxs_out.at[pl.ds(pl.multiple_of(M, 8), CAP - M)])

  def offs(ci, run):
    sl = pl.ds(ci * 16, 16)
    tot = jnp.zeros((16,), jnp.int32)
    part = jnp.zeros((16,), jnp.int32)
    for s in range(NW):
      row = hist_v[s, sl]
      tot = tot + row
      pre = jnp.where(s < wid, jnp.int32(1), jnp.int32(0))
      part = part + row * pre
    csum = plsc.cumsum(tot)
    off_v[sl] = csum - tot + run + part
    return run + jnp.sum(tot)
  lax.fori_loop(0, BIN_PAD // 16, offs, 0)

  for j in range(4):
    for r in range(8):
      def place(tt, _, r=r, j=j):
        t = j * 64 + r * 8 + tt
        v = pid_v[pl.ds(t * 16, 16)]
        bins = lax.shift_right_logical(v, sh8)
        rank, last = _dup_ranks(bins)
        cur = plsc.load_gather(off_v, [bins])
        plsc.store_scatter(off_v, [bins], cur + rank + 1, mask=last)
        dst_v[r, pl.ds(tt * 16, 16)] = cur + rank
        idx_v[pl.ds(r * 128 + tt * 16, 16)] = (
            wid * CHUNK + j * 1024 + r * 128 + tt * 16 + lane)
        return 0
      lax.fori_loop(0, 8, place, 0)
    cps = []
    for r in range(8):
      cps.append(pltpu.async_copy(
          idx_v.at[pl.ds(r * 128, 128)], idxs_out.at[dst_v.at[r]], sem))
      cps.append(pltpu.async_copy(
          pid_v.at[pl.ds(j * 1024 + r * 128, 128)], pids_out.at[dst_v.at[r]],
          sem))
    for cp in cps:
      cp.wait()


def _k5_body(pids_hbm, idxs_hbm, xlin_hbm, hist_hbm,
             mx_out, mn_out, cnts_out, part_out,
             hist_v, starts_v, tot_v, mx_v, mn_v, cnt_v, pid_st, idx_st, xg,
             acc_v, sem):
  wid = lax.axis_index("s") * 2 + lax.axis_index("c")
  pltpu.sync_copy(hist_hbm, hist_v)

  def offs(ci, run):
    sl = pl.ds(ci * 16, 16)
    tot = jnp.zeros((16,), jnp.int32)
    for s in range(NW):
      tot = tot + hist_v[s, sl]
    csum = plsc.cumsum(tot)
    starts_v[sl] = csum - tot + run
    tot_v[sl] = tot
    return run + jnp.sum(tot)
  lax.fori_loop(0, BIN_PAD // 16, offs, 0)

  lane = lax.broadcasted_iota(jnp.int32, (16,), 0)
  fz = jnp.zeros((16,), jnp.float32)
  fo = jnp.full((16,), 1.0, jnp.float32)
  sums = (fz, fz, fz, fz, fz)

  for bi in range(NBINS // NW):
    b = wid * (NBINS // NW) + bi
    start = starts_v[pl.ds(b, 16)][0]
    cnt = tot_v[pl.ds(b, 16)][0]
    for k in range(272 // 16):
      cnt_v[pl.ds(k * 16, 16)] = jnp.zeros((16,), jnp.int32)

    nwin = lax.shift_right_logical(cnt + 511, 9)

    def win(w, sums, start=start, cnt=cnt):
      base = start + w * 512
      n = jnp.minimum(512, cnt - w * 512)
      a0 = jnp.bitwise_and(base, -8)
      skip = base - a0
      pltpu.sync_copy(pids_hbm.at[pl.ds(pl.multiple_of(a0, 8), 528)], pid_st)
      pltpu.sync_copy(idxs_hbm.at[pl.ds(pl.multiple_of(a0, 8), 640)], idx_st)
      cps = []
      for r in range(5):
        cps.append(pltpu.async_copy(
            xlin_hbm.at[idx_st.at[pl.ds(r * 128, 128)]],
            xg.at[pl.ds(r * 128, 128)], sem))
      for cp in cps:
        cp.wait()

      def pt(i, sums):
        s1lo, s1hi, s2lo, s2hi, occv = sums
        pv = pid_st[pl.ds(skip + i, 16)][0]
        col = jnp.bitwise_and(pv, 255)
        rv = cnt_v[pl.ds(col, 16)]
        r = rv[0]
        plsc.store_scatter(cnt_v, [jnp.full((16,), col, jnp.int32)], rv + 1,
                           mask=lane == 0)
        gi16 = jnp.full((16,), skip + i, jnp.int32)
        x0 = plsc.load_gather(xg, [gi16, lane])
        x1 = plsc.load_gather(xg, [gi16, lane + 16])
        kf = jnp.where(r < T, jnp.float32(1.0), jnp.float32(0.0))
        s1lo = s1lo + x0 * kf
        s1hi = s1hi + x1 * kf
        s2lo = s2lo + x0 * x0 * kf
        s2hi = s2hi + x1 * x1 * kf

        rfull = jnp.full((16,), r, jnp.int32)
        keepm = rfull < T
        firstm = rfull == 0
        mxlo = mx_v[pl.ds(col * 32, 16)]
        mxhi = mx_v[pl.ds(col * 32 + 16, 16)]
        mnlo = mn_v[pl.ds(col * 32, 16)]
        mnhi = mn_v[pl.ds(col * 32 + 16, 16)]
        mx_v[pl.ds(col * 32, 16)] = jnp.where(
            keepm, jnp.where(firstm, x0, jnp.maximum(mxlo, x0)), mxlo)
        mx_v[pl.ds(col * 32 + 16, 16)] = jnp.where(
            keepm, jnp.where(firstm, x1, jnp.maximum(mxhi, x1)), mxhi)
        mn_v[pl.ds(col * 32, 16)] = jnp.where(
            keepm, jnp.where(firstm, x0, jnp.minimum(mnlo, x0)), mnlo)
        mn_v[pl.ds(col * 32 + 16, 16)] = jnp.where(
            keepm, jnp.where(firstm, x1, jnp.minimum(mnhi, x1)), mnhi)

        return (s1lo, s1hi, s2lo, s2hi, occv)

      return lax.fori_loop(0, n, pt, sums)

    sums = lax.fori_loop(0, nwin, win, sums)

    s1lo, s1hi, s2lo, s2hi, occv = sums
    for k in range(256 // 16):
      ch = cnt_v[pl.ds(k * 16, 16)]
      occv = occv + jnp.where(ch > 0, fo, fz)
    sums = (s1lo, s1hi, s2lo, s2hi, occv)

    pltpu.sync_copy(cnt_v.at[pl.ds(0, 256)], cnts_out.at[pl.ds(pl.multiple_of(b * 256, 8), 256)])
    pltpu.sync_copy(mx_v, mx_out.at[pl.ds(pl.multiple_of(b * 8192, 8), 8192)])
    pltpu.sync_copy(mn_v, mn_out.at[pl.ds(pl.multiple_of(b * 8192, 8), 8192)])

  s1lo, s1hi, s2lo, s2hi, occv = sums
  acc_v[pl.ds(0, 16)] = s1lo
  acc_v[pl.ds(16, 16)] = s1hi
  acc_v[pl.ds(32, 16)] = s2lo
  acc_v[pl.ds(48, 16)] = s2hi
  acc_v[pl.ds(64, 16)] = occv
  pltpu.sync_copy(acc_v, part_out.at[pl.ds(pl.multiple_of(wid * 80, 8), 80)])


def _k5b_body(part_ref, g_ref, b_ref, ab_ref):
  S = part_ref[...]
  S1 = jnp.sum(S[:, 0:32], axis=0, keepdims=True)
  S2 = jnp.sum(S[:, 32:64], axis=0, keepdims=True)
  Pocc = jnp.sum(S[:, 64:80])
  denom = Pocc * jnp.float32(T)
  mean = S1 / denom
  var = S2 / denom - mean * mean
  a = g_ref[...] / jnp.sqrt(var + EPS)
  b2 = b_ref[...] - mean * a
  ab_ref[...] = jnp.transpose(
      jnp.concatenate([a, b2, jnp.zeros((6, C), jnp.float32)], axis=0))


def _k6_body(mx_ref, mn_ref, cnt_ref, ab_ref, out_ref):
  mxT = jnp.transpose(mx_ref[...])
  mnT = jnp.transpose(mn_ref[...])
  cvec = cnt_ref[0]
  has_empty = cvec < T
  occ = cvec > 0
  a = ab_ref[:, 0:1]
  b2 = ab_ref[:, 1:2]
  mxT = jnp.where(has_empty, jnp.maximum(mxT, 0.0), mxT)
  mnT = jnp.where(has_empty, jnp.minimum(mnT, 0.0), mnT)
  e = jnp.where(a >= 0.0, a * mxT + b2, a * mnT + b2)
  e = jnp.maximum(e, 0.0)
  e = jnp.where(occ, e, 0.0)
  out_ref[...] = e.reshape(1, C, 8, GW)


def kernel(batch_points, W_lin, gamma, beta):
  pts = jnp.transpose(batch_points.reshape(M, 6))

  rmax = pl.pallas_call(
      _k1_body,
      out_shape=jax.ShapeDtypeStruct((1, 1), jnp.float32),
  )(pts)

  pid, xlin = pl.pallas_call(
      _k2_body,
      grid=(M // PB,),
      in_specs=[
          pl.BlockSpec((6, PB), lambda i: (0, i)),
          pl.BlockSpec((C, FIN), lambda i: (0, 0)),
          pl.BlockSpec((1, 1), lambda i: (0, 0)),
      ],
      out_specs=[
          pl.BlockSpec((PB,), lambda i: (i,)),
          pl.BlockSpec((PB, C), lambda i: (i, 0)),
      ],
      out_shape=[
          jax.ShapeDtypeStruct((M,), jnp.int32),
          jax.ShapeDtypeStruct((M, C), jnp.float32),
      ],
  )(pts, W_lin, rmax)

  hist = pl.kernel(
      _k3_body,
      out_type=jax.ShapeDtypeStruct((NW, BIN_PAD), jnp.int32),
      mesh=_mesh,
      scratch_types=[
          pltpu.VMEM((CHUNK,), jnp.int32),
          pltpu.VMEM((BIN_PAD,), jnp.int32),
      ],
      compiler_params=_sc_params,
  )(pid)

  pids_s, idxs_s = pl.kernel(
      _k4_body,
      out_type=(
          jax.ShapeDtypeStruct((CAP,), jnp.int32),
          jax.ShapeDtypeStruct((CAP,), jnp.int32),
      ),
      mesh=_mesh,
      scratch_types=[
          pltpu.VMEM((CHUNK,), jnp.int32),
          pltpu.VMEM((NW, BIN_PAD), jnp.int32),
          pltpu.VMEM((BIN_PAD,), jnp.int32),
          pltpu.VMEM((8, 128), jnp.int32),
          pltpu.VMEM((1024,), jnp.int32),
          pltpu.SemaphoreType.DMA,
      ],
      compiler_params=_sc_params,
  )(pid, hist)

  mx, mn, cnts, part = pl.kernel(
      _k5_body,
      out_type=(
          jax.ShapeDtypeStruct((Ptot * C,), jnp.float32),
          jax.ShapeDtypeStruct((Ptot * C,), jnp.float32),
          jax.ShapeDtypeStruct((NBINS * GW,), jnp.int32),
          jax.ShapeDtypeStruct((NW * 80,), jnp.float32),
      ),
      mesh=_mesh,
      scratch_types=[
          pltpu.VMEM((NW, BIN_PAD), jnp.int32),
          pltpu.VMEM((BIN_PAD,), jnp.int32),
          pltpu.VMEM((BIN_PAD,), jnp.int32),
          pltpu.VMEM((GW * C,), jnp.float32),
          pltpu.VMEM((GW * C,), jnp.float32),
          pltpu.VMEM((272,), jnp.int32),
          pltpu.VMEM((528,), jnp.int32),
          pltpu.VMEM((640,), jnp.int32),
          pltpu.VMEM((640, C), jnp.float32),
          pltpu.VMEM((80,), jnp.float32),
          pltpu.SemaphoreType.DMA,
      ],
      compiler_params=_sc_params,
  )(pids_s, idxs_s, xlin, hist)

  ab = pl.pallas_call(
      _k5b_body,
      out_shape=jax.ShapeDtypeStruct((C, 8), jnp.float32),
  )(part.reshape(NW, 80), gamma.reshape(1, C), beta.reshape(1, C))

  bev = pl.pallas_call(
      _k6_body,
      grid=(B, GH // 8),
      in_specs=[
          pl.BlockSpec((8 * GW, C), lambda b, gj: (b * (GH // 8) + gj, 0)),
          pl.BlockSpec((8 * GW, C), lambda b, gj: (b * (GH // 8) + gj, 0)),
          pl.BlockSpec((1, 1, 8 * GW), lambda b, gj: (b * (GH // 8) + gj, 0, 0)),
          pl.BlockSpec((C, 8), lambda b, gj: (0, 0)),
      ],
      out_specs=pl.BlockSpec((1, C, 8, GW), lambda b, gj: (b, 0, gj, 0)),
      out_shape=jax.ShapeDtypeStruct((B, C, GH, GW), jnp.float32),
  )(mx.reshape(Ptot, C), mn.reshape(Ptot, C),
    cnts.reshape(NBINS // 8, 1, 8 * GW), ab)

  return bev


# speedup vs baseline: 1.4401x; 1.4401x over previous
import functools

import jax
import jax.numpy as jnp
from jax import lax
from jax.experimental import pallas as pl
from jax.experimental.pallas import tpu as pltpu
from jax.experimental.pallas import tpu_sc as plsc

B, N = 4, 32768
M = B * N
GH, GW = 256, 256
T = 16
FIN = 10
C = 32
Ptot = B * GH * GW
VX = 1.0 / GW
VY = 1.0 / GH
ZC = 0.5
EPS = 1e-5

NW = 32
CHUNK = M // NW
NBINS = B * GH
BIN_PAD = 1040
CAP = M + 704
PB = 8192

_mesh = plsc.VectorSubcoreMesh(core_axis_name="c", subcore_axis_name="s")
_sc_params = pltpu.CompilerParams(
    needs_layout_passes=False, use_tc_tiling_on_sc=False)


def _dup_ranks(v):
  lane = lax.broadcasted_iota(jnp.int32, (16,), 0)
  one = jnp.full((16,), 1, jnp.int32)
  zero = jnp.zeros((16,), jnp.int32)
  rank = jnp.zeros((16,), jnp.int32)
  total = jnp.zeros((16,), jnp.int32)
  for j in range(16):
    eqm = v == v[j]
    rank = rank + jnp.where(eqm & (lane > j), one, zero)
    total = total + jnp.where(eqm, one, zero)
  return rank, rank == total - 1


def _k1_body(pts_ref, out_ref):
  x = pts_ref[3:4, :]
  y = pts_ref[4:5, :]
  z = pts_ref[5:6, :]
  gi = (x * GW).astype(jnp.int32)
  gj = (y * GH).astype(jnp.int32)
  valid = (gi >= 0) & (gi < GW) & (gj >= 0) & (gj < GH) & (z >= 0.0) & (z < 1.0)
  rgb = pts_ref[0:3, :]
  out_ref[...] = jnp.max(jnp.where(valid, rgb, -jnp.inf)).reshape(1, 1)


def _k2_body(pts_ref, w_ref, rmax_ref, pid_ref, xlin_ref):
  i = pl.program_id(0)
  x = pts_ref[3:4, :]
  y = pts_ref[4:5, :]
  z = pts_ref[5:6, :]
  gi = (x * GW).astype(jnp.int32)
  gj = (y * GH).astype(jnp.int32)
  valid = (gi >= 0) & (gi < GW) & (gj >= 0) & (gj < GH) & (z >= 0.0) & (z < 1.0)
  gidx = i * PB + lax.broadcasted_iota(jnp.int32, (1, PB), 1)
  bidx = lax.shift_right_logical(gidx, 15)
  pid = jnp.where(valid, bidx * (GH * GW) + gj * GW + gi, Ptot)
  pid_ref[...] = pid.reshape(PB)

  scale = jnp.where(rmax_ref[0, 0] > 1.1, jnp.float32(1.0 / 255.0),
                    jnp.float32(1.0))
  cxp = (gi.astype(jnp.float32) + 0.5) * VX
  cyp = (gj.astype(jnp.float32) + 0.5) * VY
  r = pts_ref[0:1, :] * scale
  g = pts_ref[1:2, :] * scale
  bch = pts_ref[2:3, :] * scale
  ones = jnp.ones((1, PB), jnp.float32)
  featsT = jnp.concatenate(
      [x, y, z, r, g, bch, x - cxp, y - cyp, z - ZC, ones], axis=0)
  feats = jnp.transpose(featsT)
  xlin_ref[...] = lax.dot_general(
      feats, w_ref[...], (((1,), (1,)), ((), ())),
      preferred_element_type=jnp.float32)


PCH = M // 16


def _k345_body(pid_hbm, xlin_hbm, mx_out, mn_out, cnts_out, part_out,
               pid_v, hist_v, off_v, tot_v, starts_v, dst_v, rec_v,
               mx_v, mn_v, cnt_v, rec_st, idx_st, xg, acc_v,
               sh_hist, sh_rec, sem):
  cid = lax.axis_index("c")
  sid = lax.axis_index("s")
  lane = lax.broadcasted_iota(jnp.int32, (16,), 0)
  sh8 = jnp.full((16,), 8, jnp.int32)
  pltpu.sync_copy(pid_hbm.at[pl.ds(pl.multiple_of(sid * PCH, 8), PCH)], pid_v)

  def zero(i, _):
    off_v[pl.ds(i * 16, 16)] = jnp.zeros((16,), jnp.int32)
    return 0
  lax.fori_loop(0, BIN_PAD // 16, zero, 0)

  def hstep(t, _):
    v = pid_v[pl.ds(t * 16, 16)]
    bins = lax.shift_right_logical(v, sh8)
    rank, last = _dup_ranks(bins)
    cur = plsc.load_gather(off_v, [bins])
    plsc.store_scatter(off_v, [bins], cur + rank + 1, mask=last)
    return 0
  lax.fori_loop(0, PCH // 16, hstep, 0)
  pltpu.sync_copy(off_v, sh_hist.at[sid])
  plsc.subcore_barrier()
  pltpu.sync_copy(sh_hist, hist_v)

  def offs(ci, run):
    sl = pl.ds(ci * 16, 16)
    tot = jnp.zeros((16,), jnp.int32)
    part = jnp.zeros((16,), jnp.int32)
    for s in range(16):
      row = hist_v[s, sl]
      tot = tot + row
      pre = jnp.where(s < sid, jnp.int32(1), jnp.int32(0))
      part = part + row * pre
    csum = plsc.cumsum(tot)
    excl = csum - tot + run
    starts_v[sl] = excl
    tot_v[sl] = tot
    off_v[sl] = excl + part
    return run + jnp.sum(tot)
  lax.fori_loop(0, BIN_PAD // 16, offs, 0)

  @pl.when(sid == 0)
  def _():
    def z(i, _):
      rec_v[pl.ds(i * 16, 16)] = jnp.zeros((16,), jnp.int32)
      return 0
    lax.fori_loop(0, (CAP - M) // 16, z, 0)
    pltpu.sync_copy(rec_v.at[pl.ds(0, CAP - M)],
                    sh_rec.at[pl.ds(pl.multiple_of(M, 8), CAP - M)])

  def pchunk(j, _):
    def prow(rr, _):
      def place(tt, _):
        t = j * 64 + rr * 8 + tt
        v = pid_v[pl.ds(t * 16, 16)]
        bins = lax.shift_right_logical(v, sh8)
        rank, last = _dup_ranks(bins)
        cur = plsc.load_gather(off_v, [bins])
        plsc.store_scatter(off_v, [bins], cur + rank + 1, mask=last)
        dst_v[rr, pl.ds(tt * 16, 16)] = cur + rank
        idxv = sid * PCH + j * 1024 + rr * 128 + tt * 16 + lane
        rec_v[pl.ds(rr * 128 + tt * 16, 16)] = jnp.bitwise_or(
            lax.shift_left(idxv, sh8), jnp.bitwise_and(v, 255))
        return 0
      lax.fori_loop(0, 8, place, 0)
      return 0
    lax.fori_loop(0, 8, prow, 0)
    for r in range(8):
      pltpu.sync_copy(rec_v.at[pl.ds(r * 128, 128)], sh_rec.at[dst_v.at[r]])
    return 0
  lax.fori_loop(0, 8, pchunk, 0)
  plsc.subcore_barrier()

  fz = jnp.zeros((16,), jnp.float32)
  fo = jnp.full((16,), 1.0, jnp.float32)
  sums = (fz, fz, fz, fz, fz)

  def bin_body(bi, sums):
    b = cid * 512 + sid * 32 + bi
    start = starts_v[pl.ds(b, 16)][0]
    cnt = tot_v[pl.ds(b, 16)][0]
    for k in range(272 // 16):
      cnt_v[pl.ds(k * 16, 16)] = jnp.zeros((16,), jnp.int32)

    nwin = lax.shift_right_logical(cnt + 511, 9)

    def win(w, sums, start=start, cnt=cnt):
      base = start + w * 512
      n = jnp.minimum(512, cnt - w * 512)
      a0 = jnp.bitwise_and(base, -8)
      skip = base - a0
      pltpu.sync_copy(sh_rec.at[pl.ds(pl.multiple_of(a0, 8), 640)], rec_st)

      def shf(k, _):
        idx_st[pl.ds(k * 16, 16)] = lax.shift_right_logical(
            rec_st[pl.ds(k * 16, 16)], sh8)
        return 0
      lax.fori_loop(0, 640 // 16, shf, 0)
      cps = []
      for r in range(5):
        cps.append(pltpu.async_copy(
            xlin_hbm.at[idx_st.at[pl.ds(r * 128, 128)]],
            xg.at[pl.ds(r * 128, 128)], sem))
      for cp in cps:
        cp.wait()

      def pt(i, sums):
        s1lo, s1hi, s2lo, s2hi, occv = sums
        pv = rec_st[pl.ds(skip + i, 16)][0]
        col = jnp.bitwise_and(pv, 255)
        rv = cnt_v[pl.ds(col, 16)]
        r = rv[0]
        plsc.store_scatter(cnt_v, [jnp.full((16,), col, jnp.int32)], rv + 1,
                           mask=lane == 0)
        gi16 = jnp.full((16,), skip + i, jnp.int32)
        x0 = plsc.load_gather(xg, [gi16, lane])
        x1 = plsc.load_gather(xg, [gi16, lane + 16])
        kf = jnp.where(r < T, jnp.float32(1.0), jnp.float32(0.0))
        s1lo = s1lo + x0 * kf
        s1hi = s1hi + x1 * kf
        s2lo = s2lo + x0 * x0 * kf
        s2hi = s2hi + x1 * x1 * kf

        rfull = jnp.full((16,), r, jnp.int32)
        keepm = rfull < T
        firstm = rfull == 0
        mxlo = mx_v[pl.ds(col * 32, 16)]
        mxhi = mx_v[pl.ds(col * 32 + 16, 16)]
        mnlo = mn_v[pl.ds(col * 32, 16)]
        mnhi = mn_v[pl.ds(col * 32 + 16, 16)]
        mx_v[pl.ds(col * 32, 16)] = jnp.where(
            keepm, jnp.where(firstm, x0, jnp.maximum(mxlo, x0)), mxlo)
        mx_v[pl.ds(col * 32 + 16, 16)] = jnp.where(
            keepm, jnp.where(firstm, x1, jnp.maximum(mxhi, x1)), mxhi)
        mn_v[pl.ds(col * 32, 16)] = jnp.where(
            keepm, jnp.where(firstm, x0, jnp.minimum(mnlo, x0)), mnlo)
        mn_v[pl.ds(col * 32 + 16, 16)] = jnp.where(
            keepm, jnp.where(firstm, x1, jnp.minimum(mnhi, x1)), mnhi)

        return (s1lo, s1hi, s2lo, s2hi, occv)

      return lax.fori_loop(0, n, pt, sums)

    sums = lax.fori_loop(0, nwin, win, sums)

    s1lo, s1hi, s2lo, s2hi, occv = sums
    for k in range(256 // 16):
      ch = cnt_v[pl.ds(k * 16, 16)]
      occv = occv + jnp.where(ch > 0, fo, fz)
    sums = (s1lo, s1hi, s2lo, s2hi, occv)

    pltpu.sync_copy(cnt_v.at[pl.ds(0, 256)], cnts_out.at[pl.ds(pl.multiple_of(b * 256, 8), 256)])
    pltpu.sync_copy(mx_v, mx_out.at[pl.ds(pl.multiple_of(b * 8192, 8), 8192)])
    pltpu.sync_copy(mn_v, mn_out.at[pl.ds(pl.multiple_of(b * 8192, 8), 8192)])
    return sums

  sums = lax.fori_loop(0, 32, bin_body, sums)
  s1lo, s1hi, s2lo, s2hi, occv = sums
  acc_v[pl.ds(0, 16)] = s1lo
  acc_v[pl.ds(16, 16)] = s1hi
  acc_v[pl.ds(32, 16)] = s2lo
  acc_v[pl.ds(48, 16)] = s2hi
  acc_v[pl.ds(64, 16)] = occv
  wid = cid * 16 + sid
  pltpu.sync_copy(acc_v, part_out.at[pl.ds(pl.multiple_of(wid * 80, 8), 80)])


def _k5b_body(part_ref, g_ref, b_ref, ab_ref):
  S = part_ref[...]
  S1 = jnp.sum(S[:, 0:32], axis=0, keepdims=True)
  S2 = jnp.sum(S[:, 32:64], axis=0, keepdims=True)
  Pocc = jnp.sum(S[:, 64:80])
  denom = Pocc * jnp.float32(T)
  mean = S1 / denom
  var = S2 / denom - mean * mean
  a = g_ref[...] / jnp.sqrt(var + EPS)
  b2 = b_ref[...] - mean * a
  ab_ref[...] = jnp.transpose(
      jnp.concatenate([a, b2, jnp.zeros((6, C), jnp.float32)], axis=0))


def _k6_body(mx_ref, mn_ref, cnt_ref, ab_ref, out_ref):
  mxT = jnp.transpose(mx_ref[...])
  mnT = jnp.transpose(mn_ref[...])
  cvec = cnt_ref[0]
  has_empty = cvec < T
  occ = cvec > 0
  a = ab_ref[:, 0:1]
  b2 = ab_ref[:, 1:2]
  mxT = jnp.where(has_empty, jnp.maximum(mxT, 0.0), mxT)
  mnT = jnp.where(has_empty, jnp.minimum(mnT, 0.0), mnT)
  e = jnp.where(a >= 0.0, a * mxT + b2, a * mnT + b2)
  e = jnp.maximum(e, 0.0)
  e = jnp.where(occ, e, 0.0)
  out_ref[...] = e.reshape(1, C, 8, GW)


def kernel(batch_points, W_lin, gamma, beta):
  pts = jnp.transpose(batch_points.reshape(M, 6))

  rmax = pl.pallas_call(
      _k1_body,
      out_shape=jax.ShapeDtypeStruct((1, 1), jnp.float32),
  )(pts)

  pid, xlin = pl.pallas_call(
      _k2_body,
      grid=(M // PB,),
      in_specs=[
          pl.BlockSpec((6, PB), lambda i: (0, i)),
          pl.BlockSpec((C, FIN), lambda i: (0, 0)),
          pl.BlockSpec((1, 1), lambda i: (0, 0)),
      ],
      out_specs=[
          pl.BlockSpec((PB,), lambda i: (i,)),
          pl.BlockSpec((PB, C), lambda i: (i, 0)),
      ],
      out_shape=[
          jax.ShapeDtypeStruct((M,), jnp.int32),
          jax.ShapeDtypeStruct((M, C), jnp.float32),
      ],
  )(pts, W_lin, rmax)

  mx, mn, cnts, part = pl.kernel(
      _k345_body,
      out_type=(
          jax.ShapeDtypeStruct((Ptot * C,), jnp.float32),
          jax.ShapeDtypeStruct((Ptot * C,), jnp.float32),
          jax.ShapeDtypeStruct((NBINS * GW,), jnp.int32),
          jax.ShapeDtypeStruct((NW * 80,), jnp.float32),
      ),
      mesh=_mesh,
      scratch_types=[
          pltpu.VMEM((PCH,), jnp.int32),
          pltpu.VMEM((16, BIN_PAD), jnp.int32),
          pltpu.VMEM((BIN_PAD,), jnp.int32),
          pltpu.VMEM((BIN_PAD,), jnp.int32),
          pltpu.VMEM((BIN_PAD,), jnp.int32),
          pltpu.VMEM((8, 128), jnp.int32),
          pltpu.VMEM((1024,), jnp.int32),
          pltpu.VMEM((GW * C,), jnp.float32),
          pltpu.VMEM((GW * C,), jnp.float32),
          pltpu.VMEM((272,), jnp.int32),
          pltpu.VMEM((640,), jnp.int32),
          pltpu.VMEM((640,), jnp.int32),
          pltpu.VMEM((640, C), jnp.float32),
          pltpu.VMEM((80,), jnp.float32),
          pltpu.VMEM_SHARED((16, BIN_PAD), jnp.int32),
          pltpu.VMEM_SHARED((CAP,), jnp.int32),
          pltpu.SemaphoreType.DMA,
      ],
      compiler_params=_sc_params,
  )(pid, xlin)

  ab = pl.pallas_call(
      _k5b_body,
      out_shape=jax.ShapeDtypeStruct((C, 8), jnp.float32),
  )(part.reshape(NW, 80), gamma.reshape(1, C), beta.reshape(1, C))

  bev = pl.pallas_call(
      _k6_body,
      grid=(B, GH // 8),
      in_specs=[
          pl.BlockSpec((8 * GW, C), lambda b, gj: (b * (GH // 8) + gj, 0)),
          pl.BlockSpec((8 * GW, C), lambda b, gj: (b * (GH // 8) + gj, 0)),
          pl.BlockSpec((1, 1, 8 * GW), lambda b, gj: (b * (GH // 8) + gj, 0, 0)),
          pl.BlockSpec((C, 8), lambda b, gj: (0, 0)),
      ],
      out_specs=pl.BlockSpec((1, C, 8, GW), lambda b, gj: (b, 0, gj, 0)),
      out_shape=jax.ShapeDtypeStruct((B, C, GH, GW), jnp.float32),
  )(mx.reshape(Ptot, C), mn.reshape(Ptot, C),
    cnts.reshape(NBINS // 8, 1, 8 * GW), ab)

  return bev
